# Initial kernel scaffold; baseline (speedup 1.0000x reference)
import functools

import jax
import jax.numpy as jnp
from jax import lax
from jax.experimental import pallas as pl
from jax.experimental.pallas import tpu as pltpu
from jax.experimental.pallas import tpu_sc as plsc

N = 10000
E = 320000
D = 128
ALPHA = 0.1
BETA = 0.5

NC = 2
NS = 16
NT = NC * NS
EPT = E // NT
CHUNK = 80
NCHUNK = EPT // CHUNK
RPT = N // NS
ZROWS = 125


def _sc_mesh():
    return plsc.VectorSubcoreMesh(core_axis_name="c", subcore_axis_name="s")


def _deg_partials(dst, zeros16, ones16):
    @functools.partial(
        pl.kernel,
        out_type=jax.ShapeDtypeStruct((NC, N, 16), jnp.float32),
        mesh=_sc_mesh(),
        scratch_types=[
            pltpu.VMEM((CHUNK,), jnp.int32),
            pltpu.VMEM((CHUNK, 16), jnp.float32),
            plsc.MemoryRef((N, 16), jnp.float32, memory_space=pltpu.VMEM_SHARED),
        ],
    )
    def k(dst_hbm, z_hbm, one_hbm, out_hbm, idx_v, ones_v, deg_sh):
        c = lax.axis_index("c")
        s = lax.axis_index("s")
        pltpu.sync_copy(z_hbm, deg_sh.at[pl.ds(s * RPT, RPT)])
        pltpu.sync_copy(one_hbm, ones_v)
        plsc.subcore_barrier()
        base = (c * NS + s) * EPT

        @pl.loop(0, NCHUNK)
        def _(i):
            pltpu.sync_copy(dst_hbm.at[pl.ds(base + i * CHUNK, CHUNK)], idx_v)
            pltpu.sync_copy(ones_v, deg_sh.at[idx_v], add=True)

        plsc.subcore_barrier()
        pltpu.sync_copy(deg_sh.at[pl.ds(s * RPT, RPT)],
                        out_hbm.at[c, pl.ds(s * RPT, RPT)])

    return k(dst, zeros16, ones16)


def _scale_tc(deg16, features):
    BLK = 1000

    def body(d_ref, f_ref, h_ref, n_ref):
        degs = d_ref[0, :, 0] + d_ref[1, :, 0]
        norm = lax.rsqrt(jnp.maximum(degs, 1.0))
        h_ref[...] = f_ref[...] * norm[:, None]
        n_ref[...] = ((1.0 - ALPHA) * norm)[:, None]

    return pl.pallas_call(
        body,
        grid=(N // BLK,),
        in_specs=[
            pl.BlockSpec((NC, BLK, 16), lambda i: (0, i, 0)),
            pl.BlockSpec((BLK, D), lambda i: (i, 0)),
        ],
        out_specs=[
            pl.BlockSpec((BLK, D), lambda i: (i, 0)),
            pl.BlockSpec((BLK, 1), lambda i: (i, 0)),
        ],
        out_shape=[
            jax.ShapeDtypeStruct((N, D), jnp.float32),
            jax.ShapeDtypeStruct((N, 1), jnp.float32),
        ],
    )(deg16, features)


def _agg_partials(h, src, dst, zeros128):
    @functools.partial(
        pl.kernel,
        out_type=jax.ShapeDtypeStruct((NC, N, D), jnp.float32),
        mesh=_sc_mesh(),
        scratch_types=[
            pltpu.VMEM((CHUNK,), jnp.int32),
            pltpu.VMEM((CHUNK,), jnp.int32),
            pltpu.VMEM((CHUNK, D), jnp.float32),
            plsc.MemoryRef((N, D), jnp.float32, memory_space=pltpu.VMEM_SHARED),
            pltpu.SemaphoreType.DMA,
        ],
    )
    def k(h_hbm, src_hbm, dst_hbm, z_hbm, out_hbm, sidx, didx, rows, agg_sh, sem):
        c = lax.axis_index("c")
        s = lax.axis_index("s")

        @pl.loop(0, RPT // ZROWS)
        def _(j):
            pltpu.sync_copy(z_hbm, agg_sh.at[pl.ds(s * RPT + j * ZROWS, ZROWS)])

        plsc.subcore_barrier()
        base = (c * NS + s) * EPT

        @pl.loop(0, NCHUNK)
        def _(i):
            pltpu.sync_copy(src_hbm.at[pl.ds(base + i * CHUNK, CHUNK)], sidx)
            pltpu.sync_copy(dst_hbm.at[pl.ds(base + i * CHUNK, CHUNK)], didx)
            pltpu.async_copy(h_hbm.at[sidx], rows, sem).wait()
            pltpu.sync_copy(rows, agg_sh.at[didx], add=True)

        plsc.subcore_barrier()
        pltpu.sync_copy(agg_sh.at[pl.ds(s * RPT, RPT)],
                        out_hbm.at[c, pl.ds(s * RPT, RPT)])

    return k(h, src, dst, zeros128)


def _combine_tc(aggp, normc, init, M1, M2):
    BLK = 1000

    def body(a_ref, n_ref, i_ref, m1_ref, m2_ref, o_ref):
        h2 = (a_ref[0] + a_ref[1]) * n_ref[...]
        o_ref[...] = (
            jnp.dot(h2, m1_ref[...], preferred_element_type=jnp.float32,
                    precision=lax.Precision.HIGHEST)
            + jnp.dot(i_ref[...], m2_ref[...], preferred_element_type=jnp.float32,
                      precision=lax.Precision.HIGHEST)
        )

    return pl.pallas_call(
        body,
        grid=(N // BLK,),
        in_specs=[
            pl.BlockSpec((NC, BLK, D), lambda i: (0, i, 0)),
            pl.BlockSpec((BLK, 1), lambda i: (i, 0)),
            pl.BlockSpec((BLK, D), lambda i: (i, 0)),
            pl.BlockSpec((D, D), lambda i: (0, 0)),
            pl.BlockSpec((D, D), lambda i: (0, 0)),
        ],
        out_specs=pl.BlockSpec((BLK, D), lambda i: (i, 0)),
        out_shape=jax.ShapeDtypeStruct((N, D), jnp.float32),
    )(aggp, normc, init, M1, M2)


def kernel(features, edge_index, initial_features, W1, W2):
    src = edge_index[0]
    dst = edge_index[1]
    eye = jnp.eye(D, dtype=jnp.float32)
    M1 = (1.0 - BETA) * eye + BETA * W1.T
    M2 = ALPHA * ((1.0 - BETA) * eye + BETA * W2.T)

    zeros16 = jnp.zeros((RPT, 16), jnp.float32)
    ones16 = jnp.ones((CHUNK, 16), jnp.float32)
    zeros128 = jnp.zeros((ZROWS, D), jnp.float32)

    deg16 = _deg_partials(dst, zeros16, ones16)
    h, normc = _scale_tc(deg16, features)
    aggp = _agg_partials(h, src, dst, zeros128)
    return _combine_tc(aggp, normc, initial_features, M1, M2)


# speedup vs baseline: 4.3527x; 4.3527x over previous
import functools

import jax
import jax.numpy as jnp
from jax import lax
from jax.experimental import pallas as pl
from jax.experimental.pallas import tpu as pltpu
from jax.experimental.pallas import tpu_sc as plsc

N = 10000
E = 320000
D = 128
ALPHA = 0.1
BETA = 0.5

NC = 2
NS = 16
NT = NC * NS
EPT = E // NT
CHUNK = 80
NCHUNK = EPT // CHUNK
NP = 10240
RPT = NP // NS
ZROWS = 128


def _sc_mesh():
    return plsc.VectorSubcoreMesh(core_axis_name="c", subcore_axis_name="s")


def _deg_partials(dst, zeros16, ones16):
    @functools.partial(
        pl.kernel,
        out_type=jax.ShapeDtypeStruct((NC, NS, RPT, 16), jnp.float32),
        mesh=_sc_mesh(),
        scratch_types=[
            pltpu.VMEM((CHUNK,), jnp.int32),
            pltpu.VMEM((CHUNK, 16), jnp.float32),
            pltpu.VMEM_SHARED((NP, 16), jnp.float32),
        ],
        compiler_params=pltpu.CompilerParams(use_tc_tiling_on_sc=False),
    )
    def k(dst_hbm, z_hbm, one_hbm, out_hbm, idx_v, ones_v, deg_sh):
        c = lax.axis_index("c")
        s = lax.axis_index("s")
        pltpu.sync_copy(z_hbm, deg_sh.at[pl.ds(s * RPT, RPT)])
        pltpu.sync_copy(one_hbm, ones_v)
        plsc.subcore_barrier()
        base = (c * NS + s) * EPT

        @pl.loop(0, NCHUNK)
        def _(i):
            pltpu.sync_copy(dst_hbm.at[pl.ds(base + i * CHUNK, CHUNK)], idx_v)
            pltpu.sync_copy(ones_v, deg_sh.at[idx_v], add=True)

        plsc.subcore_barrier()
        pltpu.sync_copy(deg_sh.at[pl.ds(s * RPT, RPT)], out_hbm.at[c, s])

    return k(dst, zeros16, ones16)


def _scale_tc(deg16, features):
    BLK = 1000

    def body(d_ref, f_ref, h_ref, n_ref):
        degs = d_ref[0, :, 0] + d_ref[1, :, 0]
        norm = lax.rsqrt(jnp.maximum(degs, 1.0))
        h_ref[...] = f_ref[...] * norm[:, None]
        n_ref[...] = ((1.0 - ALPHA) * norm)[:, None]

    return pl.pallas_call(
        body,
        grid=(N // BLK,),
        in_specs=[
            pl.BlockSpec((NC, BLK, 16), lambda i: (0, i, 0)),
            pl.BlockSpec((BLK, D), lambda i: (i, 0)),
        ],
        out_specs=[
            pl.BlockSpec((BLK, D), lambda i: (i, 0)),
            pl.BlockSpec((BLK, 1), lambda i: (i, 0)),
        ],
        out_shape=[
            jax.ShapeDtypeStruct((N, D), jnp.float32),
            jax.ShapeDtypeStruct((N, 1), jnp.float32),
        ],
    )(deg16, features)


def _agg_partials(h, src, dst, zeros128):
    @functools.partial(
        pl.kernel,
        out_type=jax.ShapeDtypeStruct((NC, NS, RPT, D), jnp.float32),
        mesh=_sc_mesh(),
        scratch_types=[
            pltpu.VMEM((CHUNK,), jnp.int32),
            pltpu.VMEM((CHUNK,), jnp.int32),
            pltpu.VMEM((CHUNK, D), jnp.float32),
            pltpu.VMEM_SHARED((NP, D), jnp.float32),
            pltpu.SemaphoreType.DMA,
        ],
    )
    def k(h_hbm, src_hbm, dst_hbm, z_hbm, out_hbm, sidx, didx, rows, agg_sh, sem):
        c = lax.axis_index("c")
        s = lax.axis_index("s")

        @pl.loop(0, RPT // ZROWS)
        def _(j):
            pltpu.sync_copy(z_hbm, agg_sh.at[pl.ds(s * RPT + j * ZROWS, ZROWS)])

        plsc.subcore_barrier()
        base = (c * NS + s) * EPT

        @pl.loop(0, NCHUNK)
        def _(i):
            pltpu.sync_copy(src_hbm.at[pl.ds(base + i * CHUNK, CHUNK)], sidx)
            pltpu.sync_copy(dst_hbm.at[pl.ds(base + i * CHUNK, CHUNK)], didx)
            pltpu.async_copy(h_hbm.at[sidx], rows, sem).wait()
            pltpu.sync_copy(rows, agg_sh.at[didx], add=True)

        plsc.subcore_barrier()
        pltpu.sync_copy(agg_sh.at[pl.ds(s * RPT, RPT)], out_hbm.at[c, s])

    return k(h, src, dst, zeros128)


def _combine_tc(aggp, normc, init, M1, M2):
    BLK = 1000

    def body(a_ref, n_ref, i_ref, m1_ref, m2_ref, o_ref):
        h2 = (a_ref[0] + a_ref[1]) * n_ref[...]
        o_ref[...] = (
            jnp.dot(h2, m1_ref[...], preferred_element_type=jnp.float32,
                    precision=lax.Precision.HIGHEST)
            + jnp.dot(i_ref[...], m2_ref[...], preferred_element_type=jnp.float32,
                      precision=lax.Precision.HIGHEST)
        )

    return pl.pallas_call(
        body,
        grid=(N // BLK,),
        in_specs=[
            pl.BlockSpec((NC, BLK, D), lambda i: (0, i, 0)),
            pl.BlockSpec((BLK, 1), lambda i: (i, 0)),
            pl.BlockSpec((BLK, D), lambda i: (i, 0)),
            pl.BlockSpec((D, D), lambda i: (0, 0)),
            pl.BlockSpec((D, D), lambda i: (0, 0)),
        ],
        out_specs=pl.BlockSpec((BLK, D), lambda i: (i, 0)),
        out_shape=jax.ShapeDtypeStruct((N, D), jnp.float32),
    )(aggp, normc, init, M1, M2)


def kernel(features, edge_index, initial_features, W1, W2):
    src = edge_index[0]
    dst = edge_index[1]
    eye = jnp.eye(D, dtype=jnp.float32)
    M1 = (1.0 - BETA) * eye + BETA * W1.T
    M2 = ALPHA * ((1.0 - BETA) * eye + BETA * W2.T)

    zeros16 = jnp.zeros((RPT, 16), jnp.float32)
    ones16 = jnp.ones((CHUNK, 16), jnp.float32)
    zeros128 = jnp.zeros((ZROWS, D), jnp.float32)

    deg16 = _deg_partials(dst, zeros16, ones16).reshape(NC, NP, 16)
    h, normc = _scale_tc(deg16, features)
    aggp = _agg_partials(h, src, dst, zeros128).reshape(NC, NP, D)
    return _combine_tc(aggp, normc, initial_features, M1, M2)


# speedup vs baseline: 4.9199x; 1.1303x over previous
import functools

import jax
import jax.numpy as jnp
from jax import lax
from jax.experimental import pallas as pl
from jax.experimental.pallas import tpu as pltpu
from jax.experimental.pallas import tpu_sc as plsc

N = 10000
E = 320000
D = 128
ALPHA = 0.1
BETA = 0.5

NC = 2
NS = 16
NT = NC * NS
EPT = E // NT
CHUNK = 80
NCHUNK = EPT // CHUNK
NP = 10240
RPT = NP // NS
ZROWS = 128
CHUNK_A = 40
NCHUNK_A = EPT // CHUNK_A


def _sc_mesh():
    return plsc.VectorSubcoreMesh(core_axis_name="c", subcore_axis_name="s")


def _deg_partials(dst, zeros16, ones16):
    @functools.partial(
        pl.kernel,
        out_type=jax.ShapeDtypeStruct((NC, NS, RPT, 16), jnp.float32),
        mesh=_sc_mesh(),
        scratch_types=[
            pltpu.VMEM((NCHUNK, CHUNK), jnp.int32),
            pltpu.VMEM((CHUNK, 16), jnp.float32),
            pltpu.VMEM_SHARED((NP, 16), jnp.float32),
            pltpu.SemaphoreType.DMA,
        ],
        compiler_params=pltpu.CompilerParams(use_tc_tiling_on_sc=False),
    )
    def k(dst_hbm, z_hbm, one_hbm, out_hbm, idx_v, ones_v, deg_sh, sem):
        c = lax.axis_index("c")
        s = lax.axis_index("s")
        tid = c * NS + s
        pltpu.sync_copy(z_hbm, deg_sh.at[pl.ds(s * RPT, RPT)])
        pltpu.sync_copy(one_hbm, ones_v)
        pltpu.sync_copy(dst_hbm.at[tid], idx_v)
        plsc.subcore_barrier()

        @pl.loop(0, NCHUNK - NCHUNK % 8, step=8)
        def _(j):
            descs = [pltpu.async_copy(ones_v, deg_sh.at[idx_v.at[j + k]], sem,
                                      add=True) for k in range(8)]
            for d in descs:
                d.wait()

        descs = [pltpu.async_copy(ones_v,
                                  deg_sh.at[idx_v.at[NCHUNK - NCHUNK % 8 + k]],
                                  sem, add=True) for k in range(NCHUNK % 8)]
        for d in descs:
            d.wait()

        plsc.subcore_barrier()
        pltpu.sync_copy(deg_sh.at[pl.ds(s * RPT, RPT)], out_hbm.at[c, s])

    return k(dst, zeros16, ones16)


def _scale_tc(deg16, features):
    BLK = 1000

    def body(d_ref, f_ref, h_ref, n_ref):
        degs = d_ref[0, :, 0] + d_ref[1, :, 0]
        norm = lax.rsqrt(jnp.maximum(degs, 1.0))
        h_ref[...] = f_ref[...] * norm[:, None]
        n_ref[...] = ((1.0 - ALPHA) * norm)[:, None]

    return pl.pallas_call(
        body,
        grid=(N // BLK,),
        in_specs=[
            pl.BlockSpec((NC, BLK, 16), lambda i: (0, i, 0)),
            pl.BlockSpec((BLK, D), lambda i: (i, 0)),
        ],
        out_specs=[
            pl.BlockSpec((BLK, D), lambda i: (i, 0)),
            pl.BlockSpec((BLK, 1), lambda i: (i, 0)),
        ],
        out_shape=[
            jax.ShapeDtypeStruct((N, D), jnp.float32),
            jax.ShapeDtypeStruct((N, 1), jnp.float32),
        ],
    )(deg16, features)


def _agg_partials(h, src, dst, zeros128):
    @functools.partial(
        pl.kernel,
        out_type=jax.ShapeDtypeStruct((NC, NS, RPT, D), jnp.float32),
        mesh=_sc_mesh(),
        scratch_types=[
            pltpu.VMEM((CHUNK,), jnp.int32),
            pltpu.VMEM((CHUNK,), jnp.int32),
            pltpu.VMEM((CHUNK, D), jnp.float32),
            pltpu.VMEM_SHARED((NP, D), jnp.float32),
            pltpu.SemaphoreType.DMA,
        ],
    )
    def k(h_hbm, src_hbm, dst_hbm, z_hbm, out_hbm, sidx, didx, rows, agg_sh, sem):
        c = lax.axis_index("c")
        s = lax.axis_index("s")
        tid = c * NS + s

        @pl.loop(0, RPT // ZROWS)
        def _(j):
            pltpu.sync_copy(z_hbm, agg_sh.at[pl.ds(s * RPT + j * ZROWS, ZROWS)])

        plsc.subcore_barrier()

        @pl.loop(0, NCHUNK)
        def _(i):
            pltpu.sync_copy(src_hbm.at[tid, i], sidx)
            pltpu.sync_copy(dst_hbm.at[tid, i], didx)
            pltpu.async_copy(h_hbm.at[sidx], rows, sem).wait()
            pltpu.sync_copy(rows, agg_sh.at[didx], add=True)

        plsc.subcore_barrier()
        pltpu.sync_copy(agg_sh.at[pl.ds(s * RPT, RPT)], out_hbm.at[c, s])

    return k(h, src, dst, zeros128)


def _combine_tc(aggp, normc, init, M1, M2):
    BLK = 1000

    def body(a_ref, n_ref, i_ref, m1_ref, m2_ref, o_ref):
        h2 = (a_ref[0] + a_ref[1]) * n_ref[...]
        o_ref[...] = (
            jnp.dot(h2, m1_ref[...], preferred_element_type=jnp.float32,
                    precision=lax.Precision.HIGHEST)
            + jnp.dot(i_ref[...], m2_ref[...], preferred_element_type=jnp.float32,
                      precision=lax.Precision.HIGHEST)
        )

    return pl.pallas_call(
        body,
        grid=(N // BLK,),
        in_specs=[
            pl.BlockSpec((NC, BLK, D), lambda i: (0, i, 0)),
            pl.BlockSpec((BLK, 1), lambda i: (i, 0)),
            pl.BlockSpec((BLK, D), lambda i: (i, 0)),
            pl.BlockSpec((D, D), lambda i: (0, 0)),
            pl.BlockSpec((D, D), lambda i: (0, 0)),
        ],
        out_specs=pl.BlockSpec((BLK, D), lambda i: (i, 0)),
        out_shape=jax.ShapeDtypeStruct((N, D), jnp.float32),
    )(aggp, normc, init, M1, M2)


def kernel(features, edge_index, initial_features, W1, W2):
    src_a = edge_index[0].reshape(NT, NCHUNK, CHUNK)
    dst_a = edge_index[1].reshape(NT, NCHUNK, CHUNK)
    dst_d = dst_a
    eye = jnp.eye(D, dtype=jnp.float32)
    M1 = (1.0 - BETA) * eye + BETA * W1.T
    M2 = ALPHA * ((1.0 - BETA) * eye + BETA * W2.T)

    zeros16 = jnp.zeros((RPT, 16), jnp.float32)
    ones16 = jnp.ones((CHUNK, 16), jnp.float32)
    zeros128 = jnp.zeros((ZROWS, D), jnp.float32)

    deg16 = _deg_partials(dst_d, zeros16, ones16).reshape(NC, NP, 16)
    h, normc = _scale_tc(deg16, features)
    aggp = _agg_partials(h, src_a, dst_a, zeros128).reshape(NC, NP, D)
    return _combine_tc(aggp, normc, initial_features, M1, M2)


# speedup vs baseline: 5.7124x; 1.1611x over previous
import functools

import jax
import jax.numpy as jnp
from jax import lax
from jax.experimental import pallas as pl
from jax.experimental.pallas import tpu as pltpu
from jax.experimental.pallas import tpu_sc as plsc

N = 10000
E = 320000
D = 128
ALPHA = 0.1
BETA = 0.5

NC = 2
NS = 16
NT = NC * NS
EPT = E // NT
CHUNK = 80
NCHUNK = EPT // CHUNK
NP = 10240
RPT = NP // NS
ZROWS = 128
CH = 100
NCH = EPT // CH
RING = 20
NGRP = NCH // RING


def _sc_mesh():
    return plsc.VectorSubcoreMesh(core_axis_name="c", subcore_axis_name="s")


def _deg_partials(dst, zeros16, ones16):
    @functools.partial(
        pl.kernel,
        out_type=jax.ShapeDtypeStruct((NC, NS, RPT, 16), jnp.float32),
        mesh=_sc_mesh(),
        scratch_types=[
            pltpu.VMEM((NCHUNK, CHUNK), jnp.int32),
            pltpu.VMEM((CHUNK, 16), jnp.float32),
            pltpu.VMEM_SHARED((NP, 16), jnp.float32),
            pltpu.SemaphoreType.DMA,
        ],
        compiler_params=pltpu.CompilerParams(use_tc_tiling_on_sc=False),
    )
    def k(dst_hbm, z_hbm, one_hbm, out_hbm, idx_v, ones_v, deg_sh, sem):
        c = lax.axis_index("c")
        s = lax.axis_index("s")
        tid = c * NS + s
        pltpu.sync_copy(z_hbm, deg_sh.at[pl.ds(s * RPT, RPT)])
        pltpu.sync_copy(one_hbm, ones_v)
        pltpu.sync_copy(dst_hbm.at[tid], idx_v)
        plsc.subcore_barrier()

        @pl.loop(0, NCHUNK - NCHUNK % 8, step=8)
        def _(j):
            descs = [pltpu.async_copy(ones_v, deg_sh.at[idx_v.at[j + k]], sem,
                                      add=True) for k in range(8)]
            for d in descs:
                d.wait()

        descs = [pltpu.async_copy(ones_v,
                                  deg_sh.at[idx_v.at[NCHUNK - NCHUNK % 8 + k]],
                                  sem, add=True) for k in range(NCHUNK % 8)]
        for d in descs:
            d.wait()

        plsc.subcore_barrier()
        pltpu.sync_copy(deg_sh.at[pl.ds(s * RPT, RPT)], out_hbm.at[c, s])

    return k(dst, zeros16, ones16)


def _scale_tc(deg16, features):
    BLK = 1000

    def body(d_ref, f_ref, h_ref, n_ref):
        degs = d_ref[0, :, 0] + d_ref[1, :, 0]
        norm = lax.rsqrt(jnp.maximum(degs, 1.0))
        h_ref[...] = (f_ref[...] * norm[:, None]).astype(jnp.bfloat16)
        n_ref[...] = ((1.0 - ALPHA) * norm)[:, None]

    return pl.pallas_call(
        body,
        grid=(N // BLK,),
        in_specs=[
            pl.BlockSpec((NC, BLK, 16), lambda i: (0, i, 0)),
            pl.BlockSpec((BLK, D), lambda i: (i, 0)),
        ],
        out_specs=[
            pl.BlockSpec((BLK, D), lambda i: (i, 0)),
            pl.BlockSpec((BLK, 1), lambda i: (i, 0)),
        ],
        out_shape=[
            jax.ShapeDtypeStruct((N, D), jnp.bfloat16),
            jax.ShapeDtypeStruct((N, 1), jnp.float32),
        ],
    )(deg16, features)


def _agg_partials(h, src, dst, zeros128):
    def _widen(pk, rows_f):
        @pl.loop(0, CH)
        def _(r):
            for g in range(4):
                w = plsc.bitcast(pk.at[r, pl.ds(32 * g, 32)][...], jnp.int32)
                lo = plsc.bitcast(w << 16, jnp.float32)
                hi = plsc.bitcast(w & jnp.int32(-65536), jnp.float32)
                rows_f.at[r, pl.ds(32 * g, 16)][...] = lo
                rows_f.at[r, pl.ds(32 * g + 16, 16)][...] = hi

    @functools.partial(
        pl.kernel,
        out_type=jax.ShapeDtypeStruct((NC, NS, RPT, D), jnp.float32),
        mesh=_sc_mesh(),
        scratch_types=[
            pltpu.VMEM((2, RING, CH), jnp.int32),
            pltpu.VMEM((2, RING, CH), jnp.int32),
            pltpu.VMEM((CH, D), jnp.bfloat16),
            pltpu.VMEM((CH, D), jnp.bfloat16),
            pltpu.VMEM((CH, D), jnp.float32),
            pltpu.VMEM_SHARED((NP, D), jnp.float32),
            pltpu.SemaphoreType.DMA,
            pltpu.SemaphoreType.DMA,
            pltpu.SemaphoreType.DMA,
        ],
        compiler_params=pltpu.CompilerParams(use_tc_tiling_on_sc=False,
                                             needs_layout_passes=False),
    )
    def k(h_hbm, src_hbm, dst_hbm, z_hbm, out_hbm,
          sring, dring, pk0, pk1, rows_f, agg_sh, sem0, sem1, semr):
        c = lax.axis_index("c")
        s = lax.axis_index("s")
        tid = c * NS + s

        @pl.loop(0, RPT // ZROWS)
        def _(j):
            pltpu.sync_copy(z_hbm, agg_sh.at[pl.ds(s * RPT + j * ZROWS, ZROWS)])

        pltpu.sync_copy(src_hbm.at[tid, 0], sring.at[0])
        pltpu.sync_copy(dst_hbm.at[tid, 0], dring.at[0])
        plsc.subcore_barrier()

        pltpu.async_copy(h_hbm.at[sring.at[0, 0]], pk0, sem0)

        @pl.loop(0, NCH, step=2)
        def _(i):
            g = i // RING
            slot = g % 2
            j = i % RING

            pltpu.make_async_copy(h_hbm.at[sring.at[slot, j]], pk0, sem0).wait()

            @pl.when(jnp.logical_and(j == 0, g + 1 < NGRP))
            def _():
                pltpu.async_copy(src_hbm.at[tid, g + 1], sring.at[1 - slot], semr)
                pltpu.async_copy(dst_hbm.at[tid, g + 1], dring.at[1 - slot], semr)

            pltpu.async_copy(h_hbm.at[sring.at[slot, j + 1]], pk1, sem1)
            _widen(pk0, rows_f)
            pltpu.sync_copy(rows_f, agg_sh.at[dring.at[slot, j]], add=True)

            g2 = (i + 2) // RING
            slot2 = g2 % 2
            j2 = (i + 2) % RING

            @pl.when(jnp.logical_and(j2 == 0, g2 < NGRP))
            def _():
                pltpu.make_async_copy(src_hbm.at[tid, 0], sring.at[0], semr).wait()
                pltpu.make_async_copy(dst_hbm.at[tid, 0], dring.at[0], semr).wait()

            @pl.when(g2 < NGRP)
            def _():
                pltpu.async_copy(h_hbm.at[sring.at[slot2, j2]], pk0, sem0)

            pltpu.make_async_copy(h_hbm.at[sring.at[slot, j + 1]], pk1, sem1).wait()
            _widen(pk1, rows_f)
            pltpu.sync_copy(rows_f, agg_sh.at[dring.at[slot, j + 1]], add=True)

        plsc.subcore_barrier()
        pltpu.sync_copy(agg_sh.at[pl.ds(s * RPT, RPT)], out_hbm.at[c, s])

    return k(h, src, dst, zeros128)


def _combine_tc(aggp, normc, init, M1, M2):
    BLK = 1000

    def body(a_ref, n_ref, i_ref, m1_ref, m2_ref, o_ref):
        h2 = (a_ref[0] + a_ref[1]) * n_ref[...]
        o_ref[...] = (
            jnp.dot(h2, m1_ref[...], preferred_element_type=jnp.float32,
                    precision=lax.Precision.HIGHEST)
            + jnp.dot(i_ref[...], m2_ref[...], preferred_element_type=jnp.float32,
                      precision=lax.Precision.HIGHEST)
        )

    return pl.pallas_call(
        body,
        grid=(N // BLK,),
        in_specs=[
            pl.BlockSpec((NC, BLK, D), lambda i: (0, i, 0)),
            pl.BlockSpec((BLK, 1), lambda i: (i, 0)),
            pl.BlockSpec((BLK, D), lambda i: (i, 0)),
            pl.BlockSpec((D, D), lambda i: (0, 0)),
            pl.BlockSpec((D, D), lambda i: (0, 0)),
        ],
        out_specs=pl.BlockSpec((BLK, D), lambda i: (i, 0)),
        out_shape=jax.ShapeDtypeStruct((N, D), jnp.float32),
    )(aggp, normc, init, M1, M2)


def kernel(features, edge_index, initial_features, W1, W2):
    src_a = edge_index[0].reshape(NT, NGRP, RING, CH)
    dst_a = edge_index[1].reshape(NT, NGRP, RING, CH)
    dst_d = edge_index[1].reshape(NT, NCHUNK, CHUNK)
    eye = jnp.eye(D, dtype=jnp.float32)
    M1 = (1.0 - BETA) * eye + BETA * W1.T
    M2 = ALPHA * ((1.0 - BETA) * eye + BETA * W2.T)
    qidx = jnp.asarray(
        [32 * g + 2 * i + p for g in range(4) for p in range(2) for i in range(16)],
        dtype=jnp.int32)
    M1 = M1[qidx, :]

    zeros16 = jnp.zeros((RPT, 16), jnp.float32)
    ones16 = jnp.ones((CHUNK, 16), jnp.float32)
    zeros128 = jnp.zeros((ZROWS, D), jnp.float32)

    deg16 = _deg_partials(dst_d, zeros16, ones16).reshape(NC, NP, 16)
    h, normc = _scale_tc(deg16, features)
    aggp = _agg_partials(h, src_a, dst_a, zeros128).reshape(NC, NP, D)
    return _combine_tc(aggp, normc, initial_features, M1, M2)


# speedup vs baseline: 6.6802x; 1.1694x over previous
import functools

import jax
import jax.numpy as jnp
from jax import lax
from jax.experimental import pallas as pl
from jax.experimental.pallas import tpu as pltpu
from jax.experimental.pallas import tpu_sc as plsc

N = 10000
E = 320000
D = 128
ALPHA = 0.1
BETA = 0.5

NC = 2
NS = 16
NT = NC * NS
EPT = E // NT
CHUNK = 80
NCHUNK = EPT // CHUNK
NP = 10240
RPT = NP // NS
ZROWS = 128
CH = 100
NCH = EPT // CH
RING = 10
NGRP = NCH // RING


def _sc_mesh():
    return plsc.VectorSubcoreMesh(core_axis_name="c", subcore_axis_name="s")


def _deg_partials(dst, zeros16, ones16):
    @functools.partial(
        pl.kernel,
        out_type=jax.ShapeDtypeStruct((NC, NS, RPT, 16), jnp.float32),
        mesh=_sc_mesh(),
        scratch_types=[
            pltpu.VMEM((NCHUNK, CHUNK), jnp.int32),
            pltpu.VMEM((CHUNK, 16), jnp.float32),
            pltpu.VMEM_SHARED((NP, 16), jnp.float32),
            pltpu.SemaphoreType.DMA,
        ],
        compiler_params=pltpu.CompilerParams(use_tc_tiling_on_sc=False),
    )
    def k(dst_hbm, z_hbm, one_hbm, out_hbm, idx_v, ones_v, deg_sh, sem):
        c = lax.axis_index("c")
        s = lax.axis_index("s")
        tid = c * NS + s
        pltpu.sync_copy(z_hbm, deg_sh.at[pl.ds(s * RPT, RPT)])
        pltpu.sync_copy(one_hbm, ones_v)
        pltpu.sync_copy(dst_hbm.at[tid], idx_v)
        plsc.subcore_barrier()

        @pl.loop(0, NCHUNK - NCHUNK % 8, step=8)
        def _(j):
            descs = [pltpu.async_copy(ones_v, deg_sh.at[idx_v.at[j + k]], sem,
                                      add=True) for k in range(8)]
            for d in descs:
                d.wait()

        descs = [pltpu.async_copy(ones_v,
                                  deg_sh.at[idx_v.at[NCHUNK - NCHUNK % 8 + k]],
                                  sem, add=True) for k in range(NCHUNK % 8)]
        for d in descs:
            d.wait()

        plsc.subcore_barrier()
        pltpu.sync_copy(deg_sh.at[pl.ds(s * RPT, RPT)], out_hbm.at[c, s])

    return k(dst, zeros16, ones16)


def _scale_tc(deg16, features):
    BLK = 1000

    def body(d_ref, f_ref, h_ref, n_ref):
        degs = d_ref[0, :, 0] + d_ref[1, :, 0]
        norm = lax.rsqrt(jnp.maximum(degs, 1.0))
        h_ref[...] = (f_ref[...] * norm[:, None]).astype(jnp.bfloat16)
        n_ref[...] = ((1.0 - ALPHA) * norm)[:, None]

    return pl.pallas_call(
        body,
        grid=(N // BLK,),
        in_specs=[
            pl.BlockSpec((NC, BLK, 16), lambda i: (0, i, 0)),
            pl.BlockSpec((BLK, D), lambda i: (i, 0)),
        ],
        out_specs=[
            pl.BlockSpec((BLK, D), lambda i: (i, 0)),
            pl.BlockSpec((BLK, 1), lambda i: (i, 0)),
        ],
        out_shape=[
            jax.ShapeDtypeStruct((N, D), jnp.bfloat16),
            jax.ShapeDtypeStruct((N, 1), jnp.float32),
        ],
    )(deg16, features)


def _agg_partials(h, src, dst, zeros128):
    def _widen(pk, rows_f):
        @pl.loop(0, CH)
        def _(r):
            for g in range(4):
                w = plsc.bitcast(pk.at[r, pl.ds(32 * g, 32)][...], jnp.int32)
                lo = plsc.bitcast(w << 16, jnp.float32)
                hi = plsc.bitcast(w & jnp.int32(-65536), jnp.float32)
                rows_f.at[r, pl.ds(32 * g, 16)][...] = lo
                rows_f.at[r, pl.ds(32 * g + 16, 16)][...] = hi

    @functools.partial(
        pl.kernel,
        out_type=jax.ShapeDtypeStruct((NC, NS, RPT, D), jnp.float32),
        mesh=_sc_mesh(),
        scratch_types=[
            pltpu.VMEM((2, RING, CH), jnp.int32),
            pltpu.VMEM((2, RING, CH), jnp.int32),
            pltpu.VMEM((CH, D), jnp.bfloat16),
            pltpu.VMEM((CH, D), jnp.bfloat16),
            pltpu.VMEM((CH, D), jnp.float32),
            pltpu.VMEM((CH, D), jnp.float32),
            pltpu.VMEM_SHARED((NP, D), jnp.float32),
            pltpu.SemaphoreType.DMA,
            pltpu.SemaphoreType.DMA,
            pltpu.SemaphoreType.DMA,
            pltpu.SemaphoreType.DMA,
            pltpu.SemaphoreType.DMA,
        ],
        compiler_params=pltpu.CompilerParams(use_tc_tiling_on_sc=False,
                                             needs_layout_passes=False),
    )
    def k(h_hbm, src_hbm, dst_hbm, z_hbm, out_hbm,
          sring, dring, pk0, pk1, rf0, rf1, agg_sh,
          sem0, sem1, semr, semsc0, semsc1):
        c = lax.axis_index("c")
        s = lax.axis_index("s")
        tid = c * NS + s

        zd = [pltpu.async_copy(z_hbm,
                               agg_sh.at[pl.ds(s * RPT + j * ZROWS, ZROWS)],
                               semr)
              for j in range(RPT // ZROWS)]
        zd.append(pltpu.async_copy(src_hbm.at[tid, 0], sring.at[0], semr))
        zd.append(pltpu.async_copy(dst_hbm.at[tid, 0], dring.at[0], semr))
        for d in zd:
            d.wait()
        plsc.subcore_barrier()

        pltpu.async_copy(h_hbm.at[sring.at[0, 0]], pk0, sem0)

        @pl.loop(0, NCH, step=2)
        def _(i):
            g = i // RING
            slot = g % 2
            j = i % RING

            pltpu.make_async_copy(h_hbm.at[sring.at[slot, j]], pk0, sem0).wait()

            @pl.when(jnp.logical_and(j == 0, g + 1 < NGRP))
            def _():
                pltpu.async_copy(src_hbm.at[tid, g + 1], sring.at[1 - slot], semr)
                pltpu.async_copy(dst_hbm.at[tid, g + 1], dring.at[1 - slot], semr)

            pltpu.async_copy(h_hbm.at[sring.at[slot, j + 1]], pk1, sem1)

            @pl.when(i >= 2)
            def _():
                pltpu.make_async_copy(rf0, agg_sh.at[dring.at[0, 0]],
                                      semsc0).wait()

            _widen(pk0, rf0)
            pltpu.async_copy(rf0, agg_sh.at[dring.at[slot, j]], semsc0,
                             add=True)

            g2 = (i + 2) // RING
            slot2 = g2 % 2
            j2 = (i + 2) % RING

            @pl.when(jnp.logical_and(j2 == 0, g2 < NGRP))
            def _():
                pltpu.make_async_copy(src_hbm.at[tid, 0], sring.at[0], semr).wait()
                pltpu.make_async_copy(dst_hbm.at[tid, 0], dring.at[0], semr).wait()

            @pl.when(g2 < NGRP)
            def _():
                pltpu.async_copy(h_hbm.at[sring.at[slot2, j2]], pk0, sem0)

            pltpu.make_async_copy(h_hbm.at[sring.at[slot, j + 1]], pk1, sem1).wait()

            @pl.when(i >= 2)
            def _():
                pltpu.make_async_copy(rf1, agg_sh.at[dring.at[0, 0]],
                                      semsc1).wait()

            _widen(pk1, rf1)
            pltpu.async_copy(rf1, agg_sh.at[dring.at[slot, j + 1]], semsc1,
                             add=True)

        pltpu.make_async_copy(rf0, agg_sh.at[dring.at[0, 0]], semsc0).wait()
        pltpu.make_async_copy(rf1, agg_sh.at[dring.at[0, 0]], semsc1).wait()
        plsc.subcore_barrier()
        pltpu.sync_copy(agg_sh.at[pl.ds(s * RPT, RPT)], out_hbm.at[c, s])

    return k(h, src, dst, zeros128)


def _combine_tc(aggp, normc, init, M1, M2):
    BLK = 1000

    def body(a_ref, n_ref, i_ref, m1_ref, m2_ref, o_ref):
        h2 = (a_ref[0] + a_ref[1]) * n_ref[...]
        o_ref[...] = (
            jnp.dot(h2, m1_ref[...], preferred_element_type=jnp.float32,
                    precision=lax.Precision.HIGHEST)
            + jnp.dot(i_ref[...], m2_ref[...], preferred_element_type=jnp.float32,
                      precision=lax.Precision.HIGHEST)
        )

    return pl.pallas_call(
        body,
        grid=(N // BLK,),
        in_specs=[
            pl.BlockSpec((NC, BLK, D), lambda i: (0, i, 0)),
            pl.BlockSpec((BLK, 1), lambda i: (i, 0)),
            pl.BlockSpec((BLK, D), lambda i: (i, 0)),
            pl.BlockSpec((D, D), lambda i: (0, 0)),
            pl.BlockSpec((D, D), lambda i: (0, 0)),
        ],
        out_specs=pl.BlockSpec((BLK, D), lambda i: (i, 0)),
        out_shape=jax.ShapeDtypeStruct((N, D), jnp.float32),
    )(aggp, normc, init, M1, M2)


def kernel(features, edge_index, initial_features, W1, W2):
    src_a = edge_index[0].reshape(NT, NGRP, RING, CH)
    dst_a = edge_index[1].reshape(NT, NGRP, RING, CH)
    dst_d = edge_index[1].reshape(NT, NCHUNK, CHUNK)
    eye = jnp.eye(D, dtype=jnp.float32)
    M1 = (1.0 - BETA) * eye + BETA * W1.T
    M2 = ALPHA * ((1.0 - BETA) * eye + BETA * W2.T)
    qidx = jnp.asarray(
        [32 * g + 2 * i + p for g in range(4) for p in range(2) for i in range(16)],
        dtype=jnp.int32)
    M1 = M1[qidx, :]

    zeros16 = jnp.zeros((RPT, 16), jnp.float32)
    ones16 = jnp.ones((CHUNK, 16), jnp.float32)
    zeros128 = jnp.zeros((ZROWS, D), jnp.float32)

    deg16 = _deg_partials(dst_d, zeros16, ones16).reshape(NC, NP, 16)
    h, normc = _scale_tc(deg16, features)
    aggp = _agg_partials(h, src_a, dst_a, zeros128).reshape(NC, NP, D)
    return _combine_tc(aggp, normc, initial_features, M1, M2)


# speedup vs baseline: 9.7605x; 1.4611x over previous
import functools

import jax
import jax.numpy as jnp
from jax import lax
from jax.experimental import pallas as pl
from jax.experimental.pallas import tpu as pltpu
from jax.experimental.pallas import tpu_sc as plsc

N = 10000
E = 320000
D = 128
ALPHA = 0.1
BETA = 0.5

NC = 2
NS = 16
NT = NC * NS
EPT = E // NT
CHUNK = 80
NCHUNK = EPT // CHUNK
NP = 10240
RPT = NP // NS
ZROWS = 128
CH = 100
NCH = EPT // CH
RING = 10
NGRP = NCH // RING


def _sc_mesh():
    return plsc.VectorSubcoreMesh(core_axis_name="c", subcore_axis_name="s")


def _deg_partials(dst, zeros16, ones16):
    @functools.partial(
        pl.kernel,
        out_type=jax.ShapeDtypeStruct((NC, NS, RPT, 16), jnp.float32),
        mesh=_sc_mesh(),
        scratch_types=[
            pltpu.VMEM((NCHUNK, CHUNK), jnp.int32),
            pltpu.VMEM((CHUNK, 16), jnp.float32),
            pltpu.VMEM_SHARED((NP, 16), jnp.float32),
            pltpu.SemaphoreType.DMA,
        ],
        compiler_params=pltpu.CompilerParams(use_tc_tiling_on_sc=False),
    )
    def k(dst_hbm, z_hbm, one_hbm, out_hbm, idx_v, ones_v, deg_sh, sem):
        c = lax.axis_index("c")
        s = lax.axis_index("s")
        tid = c * NS + s
        pltpu.sync_copy(z_hbm, deg_sh.at[pl.ds(s * RPT, RPT)])
        pltpu.sync_copy(one_hbm, ones_v)
        pltpu.sync_copy(dst_hbm.at[tid], idx_v)
        plsc.subcore_barrier()

        @pl.loop(0, NCHUNK - NCHUNK % 8, step=8)
        def _(j):
            descs = [pltpu.async_copy(ones_v, deg_sh.at[idx_v.at[j + k]], sem,
                                      add=True) for k in range(8)]
            for d in descs:
                d.wait()

        descs = [pltpu.async_copy(ones_v,
                                  deg_sh.at[idx_v.at[NCHUNK - NCHUNK % 8 + k]],
                                  sem, add=True) for k in range(NCHUNK % 8)]
        for d in descs:
            d.wait()

        plsc.subcore_barrier()
        pltpu.sync_copy(deg_sh.at[pl.ds(s * RPT, RPT)], out_hbm.at[c, s])

    return k(dst, zeros16, ones16)


def _scale_tc(deg16, features):
    BLK = 1000

    def body(d_ref, f_ref, h_ref, n_ref):
        degs = d_ref[0, :, 0] + d_ref[1, :, 0]
        norm = lax.rsqrt(jnp.maximum(degs, 1.0))
        h_ref[...] = f_ref[...] * norm[:, None]
        n_ref[...] = ((1.0 - ALPHA) * norm)[:, None]

    return pl.pallas_call(
        body,
        grid=(N // BLK,),
        in_specs=[
            pl.BlockSpec((NC, BLK, 16), lambda i: (0, i, 0)),
            pl.BlockSpec((BLK, D), lambda i: (i, 0)),
        ],
        out_specs=[
            pl.BlockSpec((BLK, D), lambda i: (i, 0)),
            pl.BlockSpec((BLK, 1), lambda i: (i, 0)),
        ],
        out_shape=[
            jax.ShapeDtypeStruct((N, D), jnp.float32),
            jax.ShapeDtypeStruct((N, 1), jnp.float32),
        ],
    )(deg16, features)


def _agg_partials(h, src, dst, zeros128):
    @functools.partial(
        pl.kernel,
        out_type=jax.ShapeDtypeStruct((NC, NS, RPT, D), jnp.float32),
        mesh=_sc_mesh(),
        scratch_types=[
            pltpu.VMEM((2, RING, CH), jnp.int32),
            pltpu.VMEM((2, RING, CH), jnp.int32),
            pltpu.VMEM((CH, D), jnp.float32),
            pltpu.VMEM((CH, D), jnp.float32),
            pltpu.VMEM_SHARED((NP, D), jnp.float32),
            pltpu.SemaphoreType.DMA,
            pltpu.SemaphoreType.DMA,
            pltpu.SemaphoreType.DMA,
            pltpu.SemaphoreType.DMA,
            pltpu.SemaphoreType.DMA,
        ],
        compiler_params=pltpu.CompilerParams(use_tc_tiling_on_sc=False,
                                             needs_layout_passes=False),
    )
    def k(h_hbm, src_hbm, dst_hbm, z_hbm, out_hbm,
          sring, dring, pk0, pk1, agg_sh,
          sem0, sem1, semr, semsc0, semsc1):
        c = lax.axis_index("c")
        s = lax.axis_index("s")
        tid = c * NS + s

        zd = [pltpu.async_copy(z_hbm,
                               agg_sh.at[pl.ds(s * RPT + j * ZROWS, ZROWS)],
                               semr)
              for j in range(RPT // ZROWS)]
        zd.append(pltpu.async_copy(src_hbm.at[tid, 0], sring.at[0], semr))
        zd.append(pltpu.async_copy(dst_hbm.at[tid, 0], dring.at[0], semr))
        for d in zd:
            d.wait()
        plsc.subcore_barrier()

        pltpu.async_copy(h_hbm.at[sring.at[0, 0]], pk0, sem0)

        @pl.loop(0, NCH, step=2)
        def _(i):
            g = i // RING
            slot = g % 2
            j = i % RING

            pltpu.make_async_copy(h_hbm.at[sring.at[slot, j]], pk0, sem0).wait()

            @pl.when(jnp.logical_and(j == 0, g + 1 < NGRP))
            def _():
                pltpu.async_copy(src_hbm.at[tid, g + 1], sring.at[1 - slot], semr)
                pltpu.async_copy(dst_hbm.at[tid, g + 1], dring.at[1 - slot], semr)

            @pl.when(i >= 2)
            def _():
                pltpu.make_async_copy(pk1, agg_sh.at[dring.at[0, 0]],
                                      semsc1).wait()

            pltpu.async_copy(h_hbm.at[sring.at[slot, j + 1]], pk1, sem1)
            pltpu.async_copy(pk0, agg_sh.at[dring.at[slot, j]], semsc0,
                             add=True)

            g2 = (i + 2) // RING
            slot2 = g2 % 2
            j2 = (i + 2) % RING

            @pl.when(jnp.logical_and(j2 == 0, g2 < NGRP))
            def _():
                pltpu.make_async_copy(src_hbm.at[tid, 0], sring.at[0], semr).wait()
                pltpu.make_async_copy(dst_hbm.at[tid, 0], dring.at[0], semr).wait()

            @pl.when(g2 < NGRP)
            def _():
                pltpu.make_async_copy(pk0, agg_sh.at[dring.at[0, 0]],
                                      semsc0).wait()
                pltpu.async_copy(h_hbm.at[sring.at[slot2, j2]], pk0, sem0)

            pltpu.make_async_copy(h_hbm.at[sring.at[slot, j + 1]], pk1, sem1).wait()
            pltpu.async_copy(pk1, agg_sh.at[dring.at[slot, j + 1]], semsc1,
                             add=True)

        pltpu.make_async_copy(pk0, agg_sh.at[dring.at[0, 0]], semsc0).wait()
        pltpu.make_async_copy(pk1, agg_sh.at[dring.at[0, 0]], semsc1).wait()
        plsc.subcore_barrier()
        pltpu.sync_copy(agg_sh.at[pl.ds(s * RPT, RPT)], out_hbm.at[c, s])

    return k(h, src, dst, zeros128)


def _combine_tc(aggp, normc, init, M1, M2):
    BLK = 1000

    def body(a_ref, n_ref, i_ref, m1_ref, m2_ref, o_ref):
        h2 = (a_ref[0] + a_ref[1]) * n_ref[...]
        o_ref[...] = (
            jnp.dot(h2, m1_ref[...], preferred_element_type=jnp.float32,
                    precision=lax.Precision.HIGHEST)
            + jnp.dot(i_ref[...], m2_ref[...], preferred_element_type=jnp.float32,
                      precision=lax.Precision.HIGHEST)
        )

    return pl.pallas_call(
        body,
        grid=(N // BLK,),
        in_specs=[
            pl.BlockSpec((NC, BLK, D), lambda i: (0, i, 0)),
            pl.BlockSpec((BLK, 1), lambda i: (i, 0)),
            pl.BlockSpec((BLK, D), lambda i: (i, 0)),
            pl.BlockSpec((D, D), lambda i: (0, 0)),
            pl.BlockSpec((D, D), lambda i: (0, 0)),
        ],
        out_specs=pl.BlockSpec((BLK, D), lambda i: (i, 0)),
        out_shape=jax.ShapeDtypeStruct((N, D), jnp.float32),
    )(aggp, normc, init, M1, M2)


def kernel(features, edge_index, initial_features, W1, W2):
    src_a = edge_index[0].reshape(NT, NGRP, RING, CH)
    dst_a = edge_index[1].reshape(NT, NGRP, RING, CH)
    dst_d = edge_index[1].reshape(NT, NCHUNK, CHUNK)
    eye = jnp.eye(D, dtype=jnp.float32)
    M1 = (1.0 - BETA) * eye + BETA * W1.T
    M2 = ALPHA * ((1.0 - BETA) * eye + BETA * W2.T)

    zeros16 = jnp.zeros((RPT, 16), jnp.float32)
    ones16 = jnp.ones((CHUNK, 16), jnp.float32)
    zeros128 = jnp.zeros((ZROWS, D), jnp.float32)

    deg16 = _deg_partials(dst_d, zeros16, ones16).reshape(NC, NP, 16)
    h, normc = _scale_tc(deg16, features)
    aggp = _agg_partials(h, src_a, dst_a, zeros128).reshape(NC, NP, D)
    return _combine_tc(aggp, normc, initial_features, M1, M2)
